# Initial kernel scaffold; baseline (speedup 1.0000x reference)
import jax
import jax.numpy as jnp
from jax.experimental import pallas as pl


def kernel(x, word_vectors):
    raise NotImplementedError("write your pallas kernel here")


# speedup vs baseline: 1.7964x; 1.7964x over previous
import functools

import jax
import jax.numpy as jnp
from jax import lax
from jax.experimental import pallas as pl
from jax.experimental.pallas import tpu as pltpu
from jax.experimental.pallas import tpu_sc as plsc

D = 64
L = 128
K = 4


def _make_lookup(n_rows: int):
    info = plsc.get_sparse_core_info()
    nc, ns = info.num_cores, info.num_subcores
    nw = nc * ns
    rows_per_w = n_rows // nw
    iters = rows_per_w // K
    assert rows_per_w % K == 0

    mesh = plsc.VectorSubcoreMesh(core_axis_name="c", subcore_axis_name="s")

    @functools.partial(
        pl.kernel,
        mesh=mesh,
        out_type=jax.ShapeDtypeStruct((n_rows * L, D), jnp.float32),
        scratch_types=[
            pltpu.VMEM((K, L), jnp.int32),
            pltpu.VMEM((K * L, D), jnp.float32),
            pltpu.SemaphoreType.DMA,
        ],
        compiler_params=pltpu.CompilerParams(use_tc_tiling_on_sc=False),
    )
    def lookup(idx_hbm, table_hbm, out_hbm, idx_v, rows_v, sem):
        wid = lax.axis_index("s") * nc + lax.axis_index("c")
        row_base = wid * rows_per_w

        def body(g, carry):
            base = row_base + g * K
            pltpu.sync_copy(idx_hbm.at[pl.ds(base, K)], idx_v)
            copies = [
                pltpu.async_copy(
                    table_hbm.at[idx_v.at[j]],
                    rows_v.at[pl.ds(j * L, L)],
                    sem,
                )
                for j in range(K)
            ]
            for cp in copies:
                cp.wait()
            pltpu.sync_copy(rows_v, out_hbm.at[pl.ds(base * L, K * L)])
            return carry

        lax.fori_loop(0, iters, body, 0)

    return lookup


def kernel(x, word_vectors):
    b, h = x.shape
    n = b * h
    idx2d = x.reshape(n // L, L).astype(jnp.int32)
    out = _make_lookup(n // L)(idx2d, word_vectors)
    return out.reshape(b, h, D)


# speedup vs baseline: 2.3822x; 1.3261x over previous
import functools

import jax
import jax.numpy as jnp
from jax import lax
from jax.experimental import pallas as pl
from jax.experimental.pallas import tpu as pltpu
from jax.experimental.pallas import tpu_sc as plsc

D = 64
L = 128
K = 4
TBLK = 8192


def _tc_transpose(wt):
    d, v = wt.shape
    grid = (v + TBLK - 1) // TBLK

    def body(in_ref, out_ref):
        a = in_ref[...].T
        out_ref[:, 0:d] = a[0 : TBLK // 2, :]
        out_ref[:, d : 2 * d] = a[TBLK // 2 : TBLK, :]

    return pl.pallas_call(
        body,
        grid=(grid,),
        in_specs=[pl.BlockSpec((d, TBLK), lambda g: (0, g))],
        out_specs=pl.BlockSpec((TBLK // 2, 2 * d), lambda g: (g, 0)),
        out_shape=jax.ShapeDtypeStruct((grid * TBLK // 2, 2 * d), jnp.float32),
    )(wt)


def _make_lookup(n_rows: int, vocab_pad: int):
    info = plsc.get_sparse_core_info()
    nc, ns = info.num_cores, info.num_subcores
    nw = nc * ns
    rows_per_w = n_rows // nw
    iters = rows_per_w // K
    assert rows_per_w % K == 0

    mesh = plsc.VectorSubcoreMesh(core_axis_name="c", subcore_axis_name="s")

    @functools.partial(
        pl.kernel,
        mesh=mesh,
        out_type=jax.ShapeDtypeStruct((n_rows * L, D), jnp.float32),
        scratch_types=[
            pltpu.VMEM((K, L), jnp.int32),
            pltpu.VMEM((K * L, D), jnp.float32),
            pltpu.SemaphoreType.DMA,
        ],
        compiler_params=pltpu.CompilerParams(use_tc_tiling_on_sc=False),
    )
    def lookup(idx_hbm, table_hbm, out_hbm, idx_v, rows_v, sem):
        wid = lax.axis_index("s") * nc + lax.axis_index("c")
        row_base = wid * rows_per_w

        def body(g, carry):
            base = row_base + g * K
            pltpu.sync_copy(idx_hbm.at[pl.ds(base, K)], idx_v)
            copies = [
                pltpu.async_copy(
                    table_hbm.at[idx_v.at[j]],
                    rows_v.at[pl.ds(j * L, L)],
                    sem,
                )
                for j in range(K)
            ]
            for cp in copies:
                cp.wait()
            pltpu.sync_copy(rows_v, out_hbm.at[pl.ds(base * L, K * L)])
            return carry

        lax.fori_loop(0, iters, body, 0)

    return lookup


def kernel(x, word_vectors):
    b, h = x.shape
    vocab, d = word_vectors.shape
    n = b * h
    half = TBLK // 2
    xi = x.reshape(-1).astype(jnp.int32)
    hi = xi // TBLK
    mid = (xi // half) % 2
    lo = xi % half
    idx2d = (hi * TBLK + 2 * lo + mid).reshape(n // L, L)
    wlin = _tc_transpose(word_vectors.T)
    vocab_pad = wlin.shape[0] * 2
    out = _make_lookup(n // L, vocab_pad)(idx2d, wlin.reshape(vocab_pad, d))
    return out.reshape(b, h, d)
